# baseline (speedup 1.0000x reference)
import jax
import jax.numpy as jnp
from jax.experimental import pallas as pl
from jax.experimental.pallas import tpu as pltpu

_N = 1_000_000
_D = 64
_NCHUNK = 8
_CHUNK = _N // _NCHUNK


def _dma_body(x_hbm, o_hbm, two_vmem, copy_sems, row_sem):
    two_vmem[...] = jnp.full((8, _D), 2.0, jnp.float32)
    copies = []
    for c in range(_NCHUNK):
        cp = pltpu.make_async_copy(
            x_hbm.at[pl.ds(c * _CHUNK, _CHUNK), :],
            o_hbm.at[pl.ds(c * _CHUNK, _CHUNK), :],
            copy_sems.at[c],
        )
        cp.start()
        copies.append(cp)
    copies[0].wait()
    r1 = pltpu.make_async_copy(
        two_vmem.at[pl.ds(0, 1), :], o_hbm.at[pl.ds(1, 1), :], row_sem)
    r1.start()
    r3 = pltpu.make_async_copy(
        two_vmem.at[pl.ds(0, 1), :], o_hbm.at[pl.ds(3, 1), :], row_sem)
    r1.wait()
    r3.start()
    r3.wait()
    for cp in copies[1:]:
        cp.wait()


def kernel(x):
    return pl.pallas_call(
        _dma_body,
        in_specs=[pl.BlockSpec(memory_space=pl.ANY)],
        out_specs=pl.BlockSpec(memory_space=pl.ANY),
        out_shape=jax.ShapeDtypeStruct((_N, _D), jnp.float32),
        scratch_shapes=[
            pltpu.VMEM((8, _D), jnp.float32),
            pltpu.SemaphoreType.DMA((_NCHUNK,)),
            pltpu.SemaphoreType.DMA,
        ],
    )(x)
